# baseline (speedup 1.0000x reference)
import jax
import jax.numpy as jnp
from jax.experimental import pallas as pl


D_MODEL = 4096
NUM_EXPERTS = 64
TILE_TOK = 512


def _router_kernel(x_ref, w_ref, b_ref, out_ref):
    x = x_ref[...]
    w = w_ref[...]
    logits = jax.lax.dot_general(
        x, w,
        dimension_numbers=(((1,), (1,)), ((), ())),
        preferred_element_type=jnp.float32,
    )
    logits = logits + b_ref[...]
    m = jnp.max(logits, axis=-1, keepdims=True)
    e = jnp.exp(logits - m)
    out_ref[...] = e / jnp.sum(e, axis=-1, keepdims=True)


def kernel(x, W, b):
    n_tok = x.shape[0]
    grid = (n_tok // TILE_TOK,)
    return pl.pallas_call(
        _router_kernel,
        grid=grid,
        in_specs=[
            pl.BlockSpec((TILE_TOK, D_MODEL), lambda i: (i, 0)),
            pl.BlockSpec((NUM_EXPERTS, D_MODEL), lambda i: (0, 0)),
            pl.BlockSpec((NUM_EXPERTS,), lambda i: (0,)),
        ],
        out_specs=pl.BlockSpec((TILE_TOK, NUM_EXPERTS), lambda i: (i, 0)),
        out_shape=jax.ShapeDtypeStruct((n_tok, NUM_EXPERTS), jnp.float32),
    )(x, W, b)


# speedup vs baseline: 1.0133x; 1.0133x over previous
import jax
import jax.numpy as jnp
from jax.experimental import pallas as pl


D_MODEL = 4096
NUM_EXPERTS = 64
TILE_TOK = 1024


def _router_kernel(x_ref, w_ref, b_ref, out_ref):
    x = x_ref[...]
    w = w_ref[...]
    logits = jax.lax.dot_general(
        x, w,
        dimension_numbers=(((1,), (1,)), ((), ())),
        preferred_element_type=jnp.float32,
    )
    logits = logits + b_ref[...]
    m = jnp.max(logits, axis=-1, keepdims=True)
    e = jnp.exp(logits - m)
    out_ref[...] = e / jnp.sum(e, axis=-1, keepdims=True)


def kernel(x, W, b):
    n_tok = x.shape[0]
    grid = (n_tok // TILE_TOK,)
    return pl.pallas_call(
        _router_kernel,
        grid=grid,
        in_specs=[
            pl.BlockSpec((TILE_TOK, D_MODEL), lambda i: (i, 0)),
            pl.BlockSpec((NUM_EXPERTS, D_MODEL), lambda i: (0, 0)),
            pl.BlockSpec((NUM_EXPERTS,), lambda i: (0,)),
        ],
        out_specs=pl.BlockSpec((TILE_TOK, NUM_EXPERTS), lambda i: (i, 0)),
        out_shape=jax.ShapeDtypeStruct((n_tok, NUM_EXPERTS), jnp.float32),
    )(x, W, b)


# speedup vs baseline: 1.0144x; 1.0011x over previous
import jax
import jax.numpy as jnp
from jax.experimental import pallas as pl
from jax.experimental.pallas import tpu as pltpu


D_MODEL = 4096
NUM_EXPERTS = 64
TILE_TOK = 1024


def _router_kernel(x_ref, w_ref, b_ref, out_ref):
    x = x_ref[...]
    w = w_ref[...]
    logits = jax.lax.dot_general(
        x, w,
        dimension_numbers=(((1,), (1,)), ((), ())),
        preferred_element_type=jnp.float32,
    )
    logits = logits + b_ref[...]
    m = jnp.max(logits, axis=-1, keepdims=True)
    e = jnp.exp(logits - m)
    out_ref[...] = e / jnp.sum(e, axis=-1, keepdims=True)


def kernel(x, W, b):
    n_tok = x.shape[0]
    grid = (n_tok // TILE_TOK,)
    return pl.pallas_call(
        _router_kernel,
        grid=grid,
        in_specs=[
            pl.BlockSpec((TILE_TOK, D_MODEL), lambda i: (i, 0)),
            pl.BlockSpec((NUM_EXPERTS, D_MODEL), lambda i: (0, 0)),
            pl.BlockSpec((NUM_EXPERTS,), lambda i: (0,)),
        ],
        out_specs=pl.BlockSpec((TILE_TOK, NUM_EXPERTS), lambda i: (i, 0)),
        out_shape=jax.ShapeDtypeStruct((n_tok, NUM_EXPERTS), jnp.float32),
        compiler_params=pltpu.CompilerParams(
            dimension_semantics=("parallel",),
        ),
    )(x, W, b)
